# baseline (speedup 1.0000x reference)
import jax
import jax.numpy as jnp
from jax import lax
from jax.experimental import pallas as pl
from jax.experimental.pallas import tpu as pltpu
from jax.experimental.pallas import tpu_sc as plsc

B = 16384
NC = 2
NS = 16
NW = NC * NS
BPW = B // NW
CHUNK = 128
NCH = BPW // CHUNK
BLK = 512


def _sc_gather_body(gene_i, dis_i, var_i, gene_t, dis_t, var_t,
                    out_g, out_d, out_v, idxv, rows, sem):
    wid = lax.axis_index("s") * NC + lax.axis_index("c")
    base = wid * BPW

    for idx_hbm, tbl, out in ((gene_i, gene_t, out_g),
                              (dis_i, dis_t, out_d),
                              (var_i, var_t, out_v)):
        pltpu.sync_copy(idx_hbm.at[wid], idxv)
        cps = [pltpu.async_copy(tbl.at[idxv.at[j]],
                                rows.at[pl.ds(j * CHUNK, CHUNK)], sem)
               for j in range(NCH)]
        for cp in cps:
            cp.wait()
        pltpu.sync_copy(rows, out.at[pl.ds(base, BPW)])


def _leaky(x):
    return jnp.where(x >= 0, x, 0.01 * x)


def _mlp_body(gene_r, dis_r, var_r, vmod_r, chrom_r, refb_r, mutb_r,
              pos_r, zyg_r, af_r, aao_r, aam_r,
              ct_r, bt_r,
              w1g_r, w1d_r, w1c_r, w1v8_r, w1r_r, w1m_r, w1x_r,
              b1_r, w2_r, b2_r, w3_r, b3_r, out_r):
    f32 = jnp.float32
    ct_proj = jnp.dot(ct_r[...], w1c_r[...], preferred_element_type=f32)
    bt_proj_r = jnp.dot(bt_r[...], w1r_r[...], preferred_element_type=f32)
    bt_proj_m = jnp.dot(bt_r[...], w1m_r[...], preferred_element_type=f32)

    iota25 = lax.broadcasted_iota(jnp.int32, (BLK, 25), 1)
    iota5 = lax.broadcasted_iota(jnp.int32, (BLK, 5), 1)
    oh_c = (chrom_r[...] == iota25).astype(f32)
    oh_r = (refb_r[...] == iota5).astype(f32)
    oh_m = (mutb_r[...] == iota5).astype(f32)

    grp = lax.broadcasted_iota(jnp.int32, (BLK, 128), 1) // 16
    vmask = (grp == vmod_r[...]).astype(f32)
    var_sel = var_r[...] * vmask

    xd = jnp.concatenate([pos_r[...], zyg_r[...], af_r[...],
                          aao_r[...], aam_r[...]], axis=1)

    h1 = jnp.dot(gene_r[...], w1g_r[...], preferred_element_type=f32)
    h1 += jnp.dot(dis_r[...], w1d_r[...], preferred_element_type=f32)
    h1 += jnp.dot(var_sel, w1v8_r[...], preferred_element_type=f32)
    h1 += jnp.dot(oh_c, ct_proj, preferred_element_type=f32)
    h1 += jnp.dot(oh_r, bt_proj_r, preferred_element_type=f32)
    h1 += jnp.dot(oh_m, bt_proj_m, preferred_element_type=f32)
    h1 += jnp.dot(xd, w1x_r[...], preferred_element_type=f32)
    h1 += b1_r[...]
    h1 = _leaky(h1)

    h2 = _leaky(jnp.dot(h1, w2_r[...], preferred_element_type=f32) + b2_r[...])
    h3 = jnp.dot(h2, w3_r[...], preferred_element_type=f32) + b3_r[...]
    out_r[...] = 1.0 / (1.0 + jnp.exp(-h3))


def kernel(gene, disease, chromosome, variant, ref_base, mut_base, position,
           zygosity, allele_freq, aa_orig_props, aa_mut_props,
           gene_table, disease_table, chromosome_table, variant_table,
           base_table, fc1_w, fc1_b, fc2_w, fc2_b, fc3_w, fc3_b):
    f32 = jnp.float32

    gi = gene.reshape(NW, NCH, CHUNK)
    di = disease.reshape(NW, NCH, CHUNK)
    vi = (variant // 8).reshape(NW, NCH, CHUNK)
    vmod = (variant % 8).reshape(B, 1)
    var_t128 = variant_table.reshape(125000, 128)
    mesh = plsc.VectorSubcoreMesh(core_axis_name="c", subcore_axis_name="s")
    sc = pl.kernel(
        _sc_gather_body,
        out_type=(jax.ShapeDtypeStruct((B, 128), f32),
                  jax.ShapeDtypeStruct((B, 128), f32),
                  jax.ShapeDtypeStruct((B, 128), f32)),
        mesh=mesh,
        scratch_types=(pltpu.VMEM((NCH, CHUNK), jnp.int32),
                       pltpu.VMEM((BPW, 128), f32),
                       pltpu.SemaphoreType.DMA),
    )
    gene_rows, dis_rows, var_rows = sc(gi, di, vi, gene_table, disease_table,
                                       var_t128)

    w1 = fc1_w.T
    w1g, w1d, w1c = w1[0:128], w1[128:256], w1[256:320]
    w1v, w1r, w1m, w1x = w1[320:336], w1[336:352], w1[352:368], w1[368:419]
    w1v8 = jnp.concatenate([w1v] * 8, axis=0)
    b1 = fc1_b.reshape(1, 128)
    w2 = fc2_w.T
    b2 = fc2_b.reshape(1, 64)
    w3 = fc3_w.T
    b3 = fc3_b.reshape(1, 1)
    chrom2 = chromosome.reshape(B, 1)
    ref2 = ref_base.reshape(B, 1)
    mut2 = mut_base.reshape(B, 1)
    pos2 = position.reshape(B, 1)
    zyg2 = zygosity.reshape(B, 1)

    grid = (B // BLK,)

    def row(d):
        return pl.BlockSpec((BLK, d), lambda i: (i, 0))

    def full(shape):
        return pl.BlockSpec(shape, lambda i: (0,) * len(shape))

    out = pl.pallas_call(
        _mlp_body,
        grid=grid,
        in_specs=[row(128), row(128), row(128), row(1), row(1), row(1), row(1),
                  row(1), row(1), row(9), row(20), row(20),
                  full((25, 64)), full((5, 16)),
                  full((128, 128)), full((128, 128)), full((64, 128)),
                  full((128, 128)), full((16, 128)), full((16, 128)),
                  full((51, 128)),
                  full((1, 128)), full((128, 64)), full((1, 64)),
                  full((64, 1)), full((1, 1))],
        out_specs=pl.BlockSpec((BLK, 1), lambda i: (i, 0)),
        out_shape=jax.ShapeDtypeStruct((B, 1), f32),
    )(gene_rows, dis_rows, var_rows, vmod, chrom2, ref2, mut2,
      pos2, zyg2, allele_freq, aa_orig_props, aa_mut_props,
      chromosome_table, base_table,
      w1g, w1d, w1c, w1v8, w1r, w1m, w1x,
      b1, w2, b2, w3, b3)
    return out


# speedup vs baseline: 1.0538x; 1.0538x over previous
import jax
import jax.numpy as jnp
from jax import lax
from jax.experimental import pallas as pl
from jax.experimental.pallas import tpu as pltpu
from jax.experimental.pallas import tpu_sc as plsc

B = 16384
NC = 2
NS = 16
NW = NC * NS
BPW = B // NW
CHUNK = 128
NCH = BPW // CHUNK
VDIM = 16
BLK = 512


def _sc_gather_body(gene_i, dis_i, var_i, gene_t, dis_t, var_t,
                    out_g, out_d, out_v, idxv, rows, rows16, sem):
    wid = lax.axis_index("s") * NC + lax.axis_index("c")
    base = wid * BPW

    for idx_hbm, tbl, out, dst in ((gene_i, gene_t, out_g, rows),
                                   (dis_i, dis_t, out_d, rows),
                                   (var_i, var_t, out_v, rows16)):
        pltpu.sync_copy(idx_hbm.at[wid], idxv)
        cps = [pltpu.async_copy(tbl.at[idxv.at[j]],
                                dst.at[pl.ds(j * CHUNK, CHUNK)], sem)
               for j in range(NCH)]
        for cp in cps:
            cp.wait()
        pltpu.sync_copy(dst, out.at[pl.ds(base, BPW)])


def _leaky(x):
    return jnp.where(x >= 0, x, 0.01 * x)


def _mlp_body(gene_r, dis_r, varT_r, chrom_r, refb_r, mutb_r, dense_r,
              ct_r, bt_r,
              w1g_r, w1d_r, w1c_r, w1v_r, w1r_r, w1m_r, w1x_r,
              b1_r, w2_r, b2_r, w3_r, b3_r, out_r):
    f32 = jnp.float32

    def tdot(a, b):
        return lax.dot_general(a, b, (((0,), (0,)), ((), ())),
                               preferred_element_type=f32)

    ct_proj = jnp.dot(ct_r[...], w1c_r[...], preferred_element_type=f32)
    btp_r = jnp.dot(bt_r[...], w1r_r[...], preferred_element_type=f32)
    btp_m = jnp.dot(bt_r[...], w1m_r[...], preferred_element_type=f32)

    i25 = lax.broadcasted_iota(jnp.int32, (25, BLK), 0)
    i5 = lax.broadcasted_iota(jnp.int32, (5, BLK), 0)
    oh_c = (jnp.broadcast_to(chrom_r[...], (25, BLK)) == i25).astype(f32)
    oh_r = (jnp.broadcast_to(refb_r[...], (5, BLK)) == i5).astype(f32)
    oh_m = (jnp.broadcast_to(mutb_r[...], (5, BLK)) == i5).astype(f32)

    h1 = jnp.dot(gene_r[...], w1g_r[...], preferred_element_type=f32)
    h1 += jnp.dot(dis_r[...], w1d_r[...], preferred_element_type=f32)
    h1 += jnp.dot(varT_r[...], w1v_r[...], preferred_element_type=f32)
    h1 += tdot(oh_c, ct_proj)
    h1 += tdot(oh_r, btp_r)
    h1 += tdot(oh_m, btp_m)
    h1 += jnp.dot(dense_r[...], w1x_r[...], preferred_element_type=f32)
    h1 += b1_r[...]
    h1 = _leaky(h1)

    h2 = _leaky(jnp.dot(h1, w2_r[...], preferred_element_type=f32) + b2_r[...])
    h3 = jnp.dot(h2, w3_r[...], preferred_element_type=f32) + b3_r[...]
    out_r[...] = 1.0 / (1.0 + jnp.exp(-h3))


def kernel(gene, disease, chromosome, variant, ref_base, mut_base, position,
           zygosity, allele_freq, aa_orig_props, aa_mut_props,
           gene_table, disease_table, chromosome_table, variant_table,
           base_table, fc1_w, fc1_b, fc2_w, fc2_b, fc3_w, fc3_b):
    f32 = jnp.float32

    gi = gene.reshape(NW, NCH, CHUNK)
    di = disease.reshape(NW, NCH, CHUNK)
    vi = variant.reshape(NW, NCH, CHUNK)
    mesh = plsc.VectorSubcoreMesh(core_axis_name="c", subcore_axis_name="s")
    sc = pl.kernel(
        _sc_gather_body,
        out_type=(jax.ShapeDtypeStruct((B, 128), f32),
                  jax.ShapeDtypeStruct((B, 128), f32),
                  jax.ShapeDtypeStruct((B, VDIM), f32)),
        mesh=mesh,
        scratch_types=(pltpu.VMEM((NCH, CHUNK), jnp.int32),
                       pltpu.VMEM((BPW, 128), f32),
                       pltpu.VMEM((BPW, VDIM), f32),
                       pltpu.SemaphoreType.DMA),
        compiler_params=pltpu.CompilerParams(use_tc_tiling_on_sc=False),
    )
    gene_rows, dis_rows, var_rows = sc(gi, di, vi, gene_table, disease_table,
                                       variant_table)

    w1 = fc1_w.T
    w1g, w1d, w1c = w1[0:128], w1[128:256], w1[256:320]
    w1v, w1r, w1m, w1x = w1[320:336], w1[336:352], w1[352:368], w1[368:419]
    b1 = fc1_b.reshape(1, 128)
    w2 = fc2_w.T
    b2 = fc2_b.reshape(1, 64)
    w3 = fc3_w.T
    b3 = fc3_b.reshape(1, 1)
    dense = jnp.concatenate([position[:, None], zygosity[:, None],
                             allele_freq, aa_orig_props, aa_mut_props],
                            axis=1)

    grid = (B // BLK,)

    def row(d):
        return pl.BlockSpec((BLK, d), lambda i: (i, 0))

    def full(shape):
        return pl.BlockSpec(shape, lambda i: (0,) * len(shape))

    out = pl.pallas_call(
        _mlp_body,
        grid=grid,
        in_specs=[row(128), row(128), row(VDIM),
                  pl.BlockSpec((BLK,), lambda i: (i,)),
                  pl.BlockSpec((BLK,), lambda i: (i,)),
                  pl.BlockSpec((BLK,), lambda i: (i,)),
                  row(51),
                  full((25, 64)), full((5, 16)),
                  full((128, 128)), full((128, 128)), full((64, 128)),
                  full((16, 128)), full((16, 128)), full((16, 128)),
                  full((51, 128)),
                  full((1, 128)), full((128, 64)), full((1, 64)),
                  full((64, 1)), full((1, 1))],
        out_specs=pl.BlockSpec((BLK, 1), lambda i: (i, 0)),
        out_shape=jax.ShapeDtypeStruct((B, 1), f32),
    )(gene_rows, dis_rows, var_rows, chromosome, ref_base, mut_base, dense,
      chromosome_table, base_table,
      w1g, w1d, w1c, w1v, w1r, w1m, w1x,
      b1, w2, b2, w3, b3)
    return out


# speedup vs baseline: 1.0910x; 1.0353x over previous
import jax
import jax.numpy as jnp
from jax import lax
from jax.experimental import pallas as pl
from jax.experimental.pallas import tpu as pltpu
from jax.experimental.pallas import tpu_sc as plsc

B = 16384
NC = 2
NS = 16
NW = NC * NS
BPW = B // NW
CHUNK = 128
NCH = BPW // CHUNK
VDIM = 16
BLK = 512


def _sc_gather2_body(gene_i, dis_i, gene_t, dis_t,
                     out_g, out_d, idxv, rows, sem):
    wid = lax.axis_index("s") * NC + lax.axis_index("c")
    base = wid * BPW
    for idx_hbm, tbl, out in ((gene_i, gene_t, out_g),
                              (dis_i, dis_t, out_d)):
        pltpu.sync_copy(idx_hbm.at[wid], idxv)
        cps = [pltpu.async_copy(tbl.at[idxv.at[j]],
                                rows.at[pl.ds(j * CHUNK, CHUNK)], sem)
               for j in range(NCH)]
        for cp in cps:
            cp.wait()
        pltpu.sync_copy(rows, out.at[pl.ds(base, BPW)])


def _sc_gatherv_body(var_i, var_t, out_v, idxv, rows16, sem):
    wid = lax.axis_index("s") * NC + lax.axis_index("c")
    base = wid * BPW
    pltpu.sync_copy(var_i.at[wid], idxv)
    cps = [pltpu.async_copy(var_t.at[idxv.at[j]],
                            rows16.at[pl.ds(j * CHUNK, CHUNK)], sem)
           for j in range(NCH)]
    for cp in cps:
        cp.wait()
    pltpu.sync_copy(rows16, out_v.at[pl.ds(base, BPW)])


def _leaky(x):
    return jnp.where(x >= 0, x, 0.01 * x)


def _mlp_body(gene_r, dis_r, var_r, chrom_r, refb_r, mutb_r,
              pos_r, zyg_r, afT_r, aaoT_r, aamT_r,
              ct_r, bt_r,
              w1g_r, w1d_r, w1c_r, w1v_r, w1r_r, w1m_r, w1x_r,
              b1_r, w2_r, b2_r, w3_r, b3_r, out_r):
    f32 = jnp.float32

    def tdot(a, b):
        return lax.dot_general(a, b, (((0,), (0,)), ((), ())),
                               preferred_element_type=f32)

    ct_proj = jnp.dot(ct_r[...], w1c_r[...], preferred_element_type=f32)
    btp_r = jnp.dot(bt_r[...], w1r_r[...], preferred_element_type=f32)
    btp_m = jnp.dot(bt_r[...], w1m_r[...], preferred_element_type=f32)

    i25 = lax.broadcasted_iota(jnp.int32, (25, BLK), 0)
    i5 = lax.broadcasted_iota(jnp.int32, (5, BLK), 0)
    oh_c = (jnp.broadcast_to(chrom_r[...], (25, BLK)) == i25).astype(f32)
    oh_r = (jnp.broadcast_to(refb_r[...], (5, BLK)) == i5).astype(f32)
    oh_m = (jnp.broadcast_to(mutb_r[...], (5, BLK)) == i5).astype(f32)

    xdT = jnp.concatenate([pos_r[...].reshape(1, BLK),
                           zyg_r[...].reshape(1, BLK),
                           afT_r[...], aaoT_r[...], aamT_r[...]], axis=0)

    h1 = jnp.dot(gene_r[...], w1g_r[...], preferred_element_type=f32)
    h1 += jnp.dot(dis_r[...], w1d_r[...], preferred_element_type=f32)
    h1 += jnp.dot(var_r[...], w1v_r[...], preferred_element_type=f32)
    h1 += tdot(oh_c, ct_proj)
    h1 += tdot(oh_r, btp_r)
    h1 += tdot(oh_m, btp_m)
    h1 += tdot(xdT, w1x_r[...])
    h1 += b1_r[...]
    h1 = _leaky(h1)

    h2 = _leaky(jnp.dot(h1, w2_r[...], preferred_element_type=f32) + b2_r[...])
    h3 = jnp.dot(h2, w3_r[...], preferred_element_type=f32) + b3_r[...]
    out_r[...] = 1.0 / (1.0 + jnp.exp(-h3))


def kernel(gene, disease, chromosome, variant, ref_base, mut_base, position,
           zygosity, allele_freq, aa_orig_props, aa_mut_props,
           gene_table, disease_table, chromosome_table, variant_table,
           base_table, fc1_w, fc1_b, fc2_w, fc2_b, fc3_w, fc3_b):
    f32 = jnp.float32
    mesh = plsc.VectorSubcoreMesh(core_axis_name="c", subcore_axis_name="s")

    vi = variant.reshape(NW, NCH, CHUNK)
    scv = pl.kernel(
        _sc_gatherv_body,
        out_type=jax.ShapeDtypeStruct((B, VDIM), f32),
        mesh=mesh,
        scratch_types=(pltpu.VMEM((NCH, CHUNK), jnp.int32),
                       pltpu.VMEM((BPW, VDIM), f32),
                       pltpu.SemaphoreType.DMA),
        compiler_params=pltpu.CompilerParams(use_tc_tiling_on_sc=False),
    )
    var_rows = scv(vi, variant_table)

    gi = gene.reshape(NW, NCH, CHUNK)
    di = disease.reshape(NW, NCH, CHUNK)
    sc2 = pl.kernel(
        _sc_gather2_body,
        out_type=(jax.ShapeDtypeStruct((B, 128), f32),
                  jax.ShapeDtypeStruct((B, 128), f32)),
        mesh=mesh,
        scratch_types=(pltpu.VMEM((NCH, CHUNK), jnp.int32),
                       pltpu.VMEM((BPW, 128), f32),
                       pltpu.SemaphoreType.DMA),
    )
    gene_rows, dis_rows = sc2(gi, di, gene_table, disease_table)

    w1 = fc1_w.T
    w1g, w1d, w1c = w1[0:128], w1[128:256], w1[256:320]
    w1v, w1r, w1m, w1x = w1[320:336], w1[336:352], w1[352:368], w1[368:419]
    b1 = fc1_b.reshape(1, 128)
    w2 = fc2_w.T
    b2 = fc2_b.reshape(1, 64)
    w3 = fc3_w.T
    b3 = fc3_b.reshape(1, 1)
    afT = allele_freq.T
    aaoT = aa_orig_props.T
    aamT = aa_mut_props.T

    grid = (B // BLK,)

    def row(d):
        return pl.BlockSpec((BLK, d), lambda i: (i, 0))

    def col(d):
        return pl.BlockSpec((d, BLK), lambda i: (0, i))

    def vec():
        return pl.BlockSpec((BLK,), lambda i: (i,))

    def full(shape):
        return pl.BlockSpec(shape, lambda i: (0,) * len(shape))

    out = pl.pallas_call(
        _mlp_body,
        grid=grid,
        in_specs=[row(128), row(128), row(VDIM),
                  vec(), vec(), vec(), vec(), vec(),
                  col(9), col(20), col(20),
                  full((25, 64)), full((5, 16)),
                  full((128, 128)), full((128, 128)), full((64, 128)),
                  full((16, 128)), full((16, 128)), full((16, 128)),
                  full((51, 128)),
                  full((1, 128)), full((128, 64)), full((1, 64)),
                  full((64, 1)), full((1, 1))],
        out_specs=pl.BlockSpec((BLK, 1), lambda i: (i, 0)),
        out_shape=jax.ShapeDtypeStruct((B, 1), f32),
    )(gene_rows, dis_rows, var_rows, chromosome, ref_base, mut_base,
      position, zygosity, afT, aaoT, aamT,
      chromosome_table, base_table,
      w1g, w1d, w1c, w1v, w1r, w1m, w1x,
      b1, w2, b2, w3, b3)
    return out


# speedup vs baseline: 3.4839x; 3.1932x over previous
import jax
import jax.numpy as jnp
from jax import lax
from jax.experimental import pallas as pl
from jax.experimental.pallas import tpu as pltpu
from jax.experimental.pallas import tpu_sc as plsc

B = 16384
NC = 2
NS = 16
NW = NC * NS
BPW = B // NW
CHUNK = 128
NCH = BPW // CHUNK
VDIM = 16
BLK = 512


def _sc_gather2_body(gene_i, dis_i, gene_t, dis_t,
                     out_g, out_d, idxv, rows, sem):
    wid = lax.axis_index("s") * NC + lax.axis_index("c")
    base = wid * BPW
    for idx_hbm, tbl, out in ((gene_i, gene_t, out_g),
                              (dis_i, dis_t, out_d)):
        pltpu.sync_copy(idx_hbm.at[wid], idxv)
        cps = [pltpu.async_copy(tbl.at[idxv.at[j]],
                                rows.at[pl.ds(j * CHUNK, CHUNK)], sem)
               for j in range(NCH)]
        for cp in cps:
            cp.wait()
        pltpu.sync_copy(rows, out.at[pl.ds(base, BPW)])


VG = 16
VB = 32


def _sc_gatherv_body(var_i, var_t3, out_v, vidx, vbuf, rows16f, sem):
    wid = lax.axis_index("s") * NC + lax.axis_index("c")
    base = wid * BPW
    pltpu.sync_copy(var_i.at[wid], vidx)
    iota16 = lax.iota(jnp.int32, 16)
    r_vec = iota16 >> 3
    s_vec = iota16 & 7

    for b in range(BPW // VB):
        b0 = b * VB

        def fetch(g, c):
            vvec = vidx[pl.ds(b0 + g * VG, VG)]
            cps = []
            for u in range(VG):
                off = pl.multiple_of((vvec[u] >> 7) * 128, 128)
                cps.append(pltpu.async_copy(
                    var_t3.at[:, :, pl.ds(off, 128)],
                    vbuf.at[:, :, pl.ds((g * VG + u) * 128, 128)], sem))
            for cp in cps:
                cp.wait()
            return c

        lax.fori_loop(0, VB // VG, fetch, 0)

        def extract(g, c):
            vvec = vidx[pl.ds(b0 + g * VG, VG)]
            for u in range(VG):
                i = b0 + g * VG + u
                lane = (g * VG + u) * 128 + (vvec[u] & 127)
                lane_vec = jnp.broadcast_to(lane, (16,))
                vals = plsc.load_gather(vbuf, [r_vec, s_vec, lane_vec])
                plsc.store_scatter(rows16f, [i * VDIM + iota16], vals)
            return c

        lax.fori_loop(0, VB // VG, extract, 0)

    pltpu.sync_copy(rows16f, out_v.at[pl.ds(base * VDIM, BPW * VDIM)])


def _leaky(x):
    return jnp.where(x >= 0, x, 0.01 * x)


def _mlp_body(gene_r, dis_r, var_r, chrom_r, refb_r, mutb_r,
              pos_r, zyg_r, afT_r, aaoT_r, aamT_r,
              ct_r, bt_r,
              w1g_r, w1d_r, w1c_r, w1v_r, w1r_r, w1m_r, w1x_r,
              b1_r, w2_r, b2_r, w3_r, b3_r, out_r):
    f32 = jnp.float32

    def tdot(a, b):
        return lax.dot_general(a, b, (((0,), (0,)), ((), ())),
                               preferred_element_type=f32)

    ct_proj = jnp.dot(ct_r[...], w1c_r[...], preferred_element_type=f32)
    btp_r = jnp.dot(bt_r[...], w1r_r[...], preferred_element_type=f32)
    btp_m = jnp.dot(bt_r[...], w1m_r[...], preferred_element_type=f32)

    i25 = lax.broadcasted_iota(jnp.int32, (25, BLK), 0)
    i5 = lax.broadcasted_iota(jnp.int32, (5, BLK), 0)
    oh_c = (jnp.broadcast_to(chrom_r[...], (25, BLK)) == i25).astype(f32)
    oh_r = (jnp.broadcast_to(refb_r[...], (5, BLK)) == i5).astype(f32)
    oh_m = (jnp.broadcast_to(mutb_r[...], (5, BLK)) == i5).astype(f32)

    xdT = jnp.concatenate([pos_r[...].reshape(1, BLK),
                           zyg_r[...].reshape(1, BLK),
                           afT_r[...], aaoT_r[...], aamT_r[...]], axis=0)

    h1 = jnp.dot(gene_r[...], w1g_r[...], preferred_element_type=f32)
    h1 += jnp.dot(dis_r[...], w1d_r[...], preferred_element_type=f32)
    h1 += jnp.dot(var_r[...], w1v_r[...], preferred_element_type=f32)
    h1 += tdot(oh_c, ct_proj)
    h1 += tdot(oh_r, btp_r)
    h1 += tdot(oh_m, btp_m)
    h1 += tdot(xdT, w1x_r[...])
    h1 += b1_r[...]
    h1 = _leaky(h1)

    h2 = _leaky(jnp.dot(h1, w2_r[...], preferred_element_type=f32) + b2_r[...])
    h3 = jnp.dot(h2, w3_r[...], preferred_element_type=f32) + b3_r[...]
    out_r[...] = 1.0 / (1.0 + jnp.exp(-h3))


def kernel(gene, disease, chromosome, variant, ref_base, mut_base, position,
           zygosity, allele_freq, aa_orig_props, aa_mut_props,
           gene_table, disease_table, chromosome_table, variant_table,
           base_table, fc1_w, fc1_b, fc2_w, fc2_b, fc3_w, fc3_b):
    f32 = jnp.float32
    mesh = plsc.VectorSubcoreMesh(core_axis_name="c", subcore_axis_name="s")

    vi = variant.reshape(NW, BPW)
    var_t3 = variant_table.T.reshape(2, 8, 1000000)
    scv = pl.kernel(
        _sc_gatherv_body,
        out_type=jax.ShapeDtypeStruct((B * VDIM,), f32),
        mesh=mesh,
        scratch_types=(pltpu.VMEM((BPW,), jnp.int32),
                       pltpu.VMEM((2, 8, VB * 128), f32),
                       pltpu.VMEM((BPW * VDIM,), f32),
                       pltpu.SemaphoreType.DMA),
        compiler_params=pltpu.CompilerParams(needs_layout_passes=False),
    )
    var_rows = scv(vi, var_t3).reshape(B, VDIM)

    gi = gene.reshape(NW, NCH, CHUNK)
    di = disease.reshape(NW, NCH, CHUNK)
    sc2 = pl.kernel(
        _sc_gather2_body,
        out_type=(jax.ShapeDtypeStruct((B, 128), f32),
                  jax.ShapeDtypeStruct((B, 128), f32)),
        mesh=mesh,
        scratch_types=(pltpu.VMEM((NCH, CHUNK), jnp.int32),
                       pltpu.VMEM((BPW, 128), f32),
                       pltpu.SemaphoreType.DMA),
    )
    gene_rows, dis_rows = sc2(gi, di, gene_table, disease_table)

    w1 = fc1_w.T
    w1g, w1d, w1c = w1[0:128], w1[128:256], w1[256:320]
    w1v, w1r, w1m, w1x = w1[320:336], w1[336:352], w1[352:368], w1[368:419]
    b1 = fc1_b.reshape(1, 128)
    w2 = fc2_w.T
    b2 = fc2_b.reshape(1, 64)
    w3 = fc3_w.T
    b3 = fc3_b.reshape(1, 1)
    afT = allele_freq.T
    aaoT = aa_orig_props.T
    aamT = aa_mut_props.T

    grid = (B // BLK,)

    def row(d):
        return pl.BlockSpec((BLK, d), lambda i: (i, 0))

    def col(d):
        return pl.BlockSpec((d, BLK), lambda i: (0, i))

    def vec():
        return pl.BlockSpec((BLK,), lambda i: (i,))

    def full(shape):
        return pl.BlockSpec(shape, lambda i: (0,) * len(shape))

    out = pl.pallas_call(
        _mlp_body,
        grid=grid,
        in_specs=[row(128), row(128), row(VDIM),
                  vec(), vec(), vec(), vec(), vec(),
                  col(9), col(20), col(20),
                  full((25, 64)), full((5, 16)),
                  full((128, 128)), full((128, 128)), full((64, 128)),
                  full((16, 128)), full((16, 128)), full((16, 128)),
                  full((51, 128)),
                  full((1, 128)), full((128, 64)), full((1, 64)),
                  full((64, 1)), full((1, 1))],
        out_specs=pl.BlockSpec((BLK, 1), lambda i: (i, 0)),
        out_shape=jax.ShapeDtypeStruct((B, 1), f32),
    )(gene_rows, dis_rows, var_rows, chromosome, ref_base, mut_base,
      position, zygosity, afT, aaoT, aamT,
      chromosome_table, base_table,
      w1g, w1d, w1c, w1v, w1r, w1m, w1x,
      b1, w2, b2, w3, b3)
    return out


# speedup vs baseline: 3.4954x; 1.0033x over previous
import jax
import jax.numpy as jnp
from jax import lax
from jax.experimental import pallas as pl
from jax.experimental.pallas import tpu as pltpu
from jax.experimental.pallas import tpu_sc as plsc

B = 16384
NC = 2
NS = 16
NW = NC * NS
BPW = B // NW
CHUNK = 128
NCH = BPW // CHUNK
VDIM = 16
BLK = 512


def _sc_gather2_body(gene_i, dis_i, gene_t, dis_t,
                     out_g, out_d, idxv, rows, sem):
    wid = lax.axis_index("s") * NC + lax.axis_index("c")
    base = wid * BPW
    for idx_hbm, tbl, out in ((gene_i, gene_t, out_g),
                              (dis_i, dis_t, out_d)):
        pltpu.sync_copy(idx_hbm.at[wid], idxv)
        cps = [pltpu.async_copy(tbl.at[idxv.at[j]],
                                rows.at[pl.ds(j * CHUNK, CHUNK)], sem)
               for j in range(NCH)]
        for cp in cps:
            cp.wait()
        pltpu.sync_copy(rows, out.at[pl.ds(base, BPW)])


VG = 16
VB = 32


def _sc_gatherv_body(var_i, var_t3, gdep, out_v, vidx, vbuf, rows16, sem):
    wid = lax.axis_index("s") * NC + lax.axis_index("c")
    base = wid * BPW
    pltpu.sync_copy(var_i.at[wid], vidx)
    iota16 = lax.iota(jnp.int32, 16)
    r_vec = iota16 >> 3
    s_vec = iota16 & 7

    for b in range(BPW // VB):
        b0 = b * VB

        def fetch(g, c):
            vvec = vidx[pl.ds(b0 + g * VG, VG)]
            cps = []
            for u in range(VG):
                off = pl.multiple_of((vvec[u] >> 7) * 128, 128)
                cps.append(pltpu.async_copy(
                    var_t3.at[:, :, pl.ds(off, 128)],
                    vbuf.at[:, :, pl.ds((g * VG + u) * 128, 128)], sem))
            for cp in cps:
                cp.wait()
            return c

        lax.fori_loop(0, VB // VG, fetch, 0)

        def extract(g, c):
            vvec = vidx[pl.ds(b0 + g * VG, VG)]
            for u in range(VG):
                i = b0 + g * VG + u
                lane = (g * VG + u) * 128 + (vvec[u] & 127)
                lane_vec = jnp.broadcast_to(lane, (16,))
                vals = plsc.load_gather(vbuf, [r_vec, s_vec, lane_vec])
                plsc.store_scatter(rows16, [i * VDIM + iota16], vals)
            return c

        lax.fori_loop(0, VB // VG, extract, 0)

    pltpu.sync_copy(rows16, out_v.at[pl.ds(base * VDIM, BPW * VDIM)])


def _leaky(x):
    return jnp.where(x >= 0, x, 0.01 * x)


def _mlp1_body(gene_r, dis_r, chrom_r, refb_r, mutb_r,
               pos_r, zyg_r, afT_r, aaoT_r, aamT_r,
               ct_r, bt_r,
               w1g_r, w1d_r, w1c_r, w1r_r, w1m_r, w1x_r,
               b1_r, h1p_r):
    f32 = jnp.float32

    def tdot(a, b):
        return lax.dot_general(a, b, (((0,), (0,)), ((), ())),
                               preferred_element_type=f32)

    ct_proj = jnp.dot(ct_r[...], w1c_r[...], preferred_element_type=f32)
    btp_r = jnp.dot(bt_r[...], w1r_r[...], preferred_element_type=f32)
    btp_m = jnp.dot(bt_r[...], w1m_r[...], preferred_element_type=f32)

    i25 = lax.broadcasted_iota(jnp.int32, (25, BLK), 0)
    i5 = lax.broadcasted_iota(jnp.int32, (5, BLK), 0)
    oh_c = (jnp.broadcast_to(chrom_r[...], (25, BLK)) == i25).astype(f32)
    oh_r = (jnp.broadcast_to(refb_r[...], (5, BLK)) == i5).astype(f32)
    oh_m = (jnp.broadcast_to(mutb_r[...], (5, BLK)) == i5).astype(f32)

    xdT = jnp.concatenate([pos_r[...].reshape(1, BLK),
                           zyg_r[...].reshape(1, BLK),
                           afT_r[...], aaoT_r[...], aamT_r[...]], axis=0)

    h1 = jnp.dot(gene_r[...], w1g_r[...], preferred_element_type=f32)
    h1 += jnp.dot(dis_r[...], w1d_r[...], preferred_element_type=f32)
    h1 += tdot(oh_c, ct_proj)
    h1 += tdot(oh_r, btp_r)
    h1 += tdot(oh_m, btp_m)
    h1 += tdot(xdT, w1x_r[...])
    h1p_r[...] = h1 + b1_r[...]


def _mlp2_body(h1p_r, var_r, w1v_r, w2_r, b2_r, w3_r, b3_r, out_r):
    f32 = jnp.float32
    h1 = _leaky(h1p_r[...] +
                jnp.dot(var_r[...], w1v_r[...], preferred_element_type=f32))
    h2 = _leaky(jnp.dot(h1, w2_r[...], preferred_element_type=f32) + b2_r[...])
    h3 = jnp.dot(h2, w3_r[...], preferred_element_type=f32) + b3_r[...]
    out_r[...] = 1.0 / (1.0 + jnp.exp(-h3))


def kernel(gene, disease, chromosome, variant, ref_base, mut_base, position,
           zygosity, allele_freq, aa_orig_props, aa_mut_props,
           gene_table, disease_table, chromosome_table, variant_table,
           base_table, fc1_w, fc1_b, fc2_w, fc2_b, fc3_w, fc3_b):
    f32 = jnp.float32
    mesh = plsc.VectorSubcoreMesh(core_axis_name="c", subcore_axis_name="s")

    gi = gene.reshape(NW, NCH, CHUNK)
    di = disease.reshape(NW, NCH, CHUNK)
    sc2 = pl.kernel(
        _sc_gather2_body,
        out_type=(jax.ShapeDtypeStruct((B, 128), f32),
                  jax.ShapeDtypeStruct((B, 128), f32)),
        mesh=mesh,
        scratch_types=(pltpu.VMEM((NCH, CHUNK), jnp.int32),
                       pltpu.VMEM((BPW, 128), f32),
                       pltpu.SemaphoreType.DMA),
    )
    gene_rows, dis_rows = sc2(gi, di, gene_table, disease_table)

    vi = variant.reshape(NW, BPW)
    var_t3 = variant_table.T.reshape(2, 8, 1000000)
    scv = pl.kernel(
        _sc_gatherv_body,
        out_type=jax.ShapeDtypeStruct((B * VDIM,), f32),
        mesh=mesh,
        scratch_types=(pltpu.VMEM((BPW,), jnp.int32),
                       pltpu.VMEM((2, 8, VB * 128), f32),
                       pltpu.VMEM((BPW * VDIM,), f32),
                       pltpu.SemaphoreType.DMA),
        compiler_params=pltpu.CompilerParams(needs_layout_passes=False),
    )
    var_rows = scv(vi, var_t3, gene_rows[:8]).reshape(B, VDIM)

    w1 = fc1_w.T
    w1g, w1d, w1c = w1[0:128], w1[128:256], w1[256:320]
    w1v, w1r, w1m, w1x = w1[320:336], w1[336:352], w1[352:368], w1[368:419]
    b1 = fc1_b.reshape(1, 128)
    w2 = fc2_w.T
    b2 = fc2_b.reshape(1, 64)
    w3 = fc3_w.T
    b3 = fc3_b.reshape(1, 1)
    afT = allele_freq.T
    aaoT = aa_orig_props.T
    aamT = aa_mut_props.T

    grid = (B // BLK,)

    def row(d):
        return pl.BlockSpec((BLK, d), lambda i: (i, 0))

    def col(d):
        return pl.BlockSpec((d, BLK), lambda i: (0, i))

    def vec():
        return pl.BlockSpec((BLK,), lambda i: (i,))

    def full(shape):
        return pl.BlockSpec(shape, lambda i: (0,) * len(shape))

    h1p = pl.pallas_call(
        _mlp1_body,
        grid=grid,
        in_specs=[row(128), row(128),
                  vec(), vec(), vec(), vec(), vec(),
                  col(9), col(20), col(20),
                  full((25, 64)), full((5, 16)),
                  full((128, 128)), full((128, 128)), full((64, 128)),
                  full((16, 128)), full((16, 128)), full((51, 128)),
                  full((1, 128))],
        out_specs=pl.BlockSpec((BLK, 128), lambda i: (i, 0)),
        out_shape=jax.ShapeDtypeStruct((B, 128), f32),
    )(gene_rows, dis_rows, chromosome, ref_base, mut_base,
      position, zygosity, afT, aaoT, aamT,
      chromosome_table, base_table,
      w1g, w1d, w1c, w1r, w1m, w1x, b1)

    out = pl.pallas_call(
        _mlp2_body,
        grid=grid,
        in_specs=[row(128), row(VDIM),
                  full((16, 128)), full((128, 64)), full((1, 64)),
                  full((64, 1)), full((1, 1))],
        out_specs=pl.BlockSpec((BLK, 1), lambda i: (i, 0)),
        out_shape=jax.ShapeDtypeStruct((B, 1), f32),
    )(h1p, var_rows, w1v, w2, b2, w3, b3)
    return out


# speedup vs baseline: 4.0528x; 1.1595x over previous
import jax
import jax.numpy as jnp
from jax import lax
from jax.experimental import pallas as pl
from jax.experimental.pallas import tpu as pltpu
from jax.experimental.pallas import tpu_sc as plsc

B = 16384
NC = 2
NS = 16
NW = NC * NS
BPW = B // NW
CHUNK = 128
NCH = BPW // CHUNK
VDIM = 16
BLK = 512


def _sc_gather2_body(gene_i, dis_i, gene_t, dis_t,
                     out_g, out_d, idxv, rows, sem):
    wid = lax.axis_index("s") * NC + lax.axis_index("c")
    base = wid * BPW
    for idx_hbm, tbl, out in ((gene_i, gene_t, out_g),
                              (dis_i, dis_t, out_d)):
        pltpu.sync_copy(idx_hbm.at[wid], idxv)
        cps = [pltpu.async_copy(tbl.at[idxv.at[j]],
                                rows.at[pl.ds(j * CHUNK, CHUNK)], sem)
               for j in range(NCH)]
        for cp in cps:
            cp.wait()
        pltpu.sync_copy(rows, out.at[pl.ds(base, BPW)])


VG = 16
VB = 32


def _sc_gatherv_body(var_i, var_t3, gdep, out_v, vidx, vbuf, rows16, sem):
    wid = lax.axis_index("s") * NC + lax.axis_index("c")
    base = wid * BPW
    pltpu.sync_copy(var_i.at[wid], vidx)
    iota16 = lax.iota(jnp.int32, 16)
    r_vec = iota16 >> 3
    s_vec = iota16 & 7

    for b in range(BPW // VB):
        b0 = b * VB

        def fetch(g, c):
            vvec = vidx[pl.ds(b0 + g * VG, VG)]
            cps = []
            for u in range(VG):
                off = pl.multiple_of((vvec[u] >> 7) * 128, 128)
                cps.append(pltpu.async_copy(
                    var_t3.at[:, :, pl.ds(off, 128)],
                    vbuf.at[:, :, pl.ds((g * VG + u) * 128, 128)], sem))
            for cp in cps:
                cp.wait()
            return c

        lax.fori_loop(0, VB // VG, fetch, 0)

        def extract(g, c):
            vvec = vidx[pl.ds(b0 + g * VG, VG)]
            for u in range(VG):
                i = b0 + g * VG + u
                lane = (g * VG + u) * 128 + (vvec[u] & 127)
                lane_vec = jnp.broadcast_to(lane, (16,))
                vals = plsc.load_gather(vbuf, [r_vec, s_vec, lane_vec])
                plsc.store_scatter(rows16, [i * VDIM + iota16], vals)
            return c

        lax.fori_loop(0, VB // VG, extract, 0)

    pltpu.sync_copy(rows16, out_v.at[pl.ds(base * VDIM, BPW * VDIM)])


def _leaky(x):
    return jnp.where(x >= 0, x, 0.01 * x)


def _mlp1_body(gene_r, dis_r, chrom_r, refb_r, mutb_r,
               pos_r, zyg_r, afT_r, aaoT_r, aamT_r,
               ct_r, bt_r,
               w1g_r, w1d_r, w1c_r, w1r_r, w1m_r, w1x_r,
               b1_r, h1p_r):
    f32 = jnp.float32

    def tdot(a, b):
        return lax.dot_general(a, b, (((0,), (0,)), ((), ())),
                               preferred_element_type=f32)

    ct_proj = jnp.dot(ct_r[...], w1c_r[...], preferred_element_type=f32)
    btp_r = jnp.dot(bt_r[...], w1r_r[...], preferred_element_type=f32)
    btp_m = jnp.dot(bt_r[...], w1m_r[...], preferred_element_type=f32)

    i25 = lax.broadcasted_iota(jnp.int32, (25, BLK), 0)
    i5 = lax.broadcasted_iota(jnp.int32, (5, BLK), 0)
    oh_c = (jnp.broadcast_to(chrom_r[...], (25, BLK)) == i25).astype(f32)
    oh_r = (jnp.broadcast_to(refb_r[...], (5, BLK)) == i5).astype(f32)
    oh_m = (jnp.broadcast_to(mutb_r[...], (5, BLK)) == i5).astype(f32)

    xdT = jnp.concatenate([pos_r[...].reshape(1, BLK),
                           zyg_r[...].reshape(1, BLK),
                           afT_r[...], aaoT_r[...], aamT_r[...]], axis=0)

    h1 = jnp.dot(gene_r[...], w1g_r[...], preferred_element_type=f32)
    h1 += jnp.dot(dis_r[...], w1d_r[...], preferred_element_type=f32)
    h1 += tdot(oh_c, ct_proj)
    h1 += tdot(oh_r, btp_r)
    h1 += tdot(oh_m, btp_m)
    h1 += tdot(xdT, w1x_r[...])
    h1p_r[...] = h1 + b1_r[...]


BLK2 = 2048


def _mlp2_body(h1p_r, var_r, w1v_r, w2_r, b2_r, w3_r, b3_r, out_r):
    f32 = jnp.float32
    h1 = _leaky(h1p_r[...] +
                jnp.dot(var_r[...], w1v_r[...], preferred_element_type=f32))
    h2 = _leaky(jnp.dot(h1, w2_r[...], preferred_element_type=f32) + b2_r[...])
    h3 = lax.dot_general(w3_r[...], h2, (((0,), (1,)), ((), ())),
                         preferred_element_type=f32) + b3_r[...]
    out_r[...] = 1.0 / (1.0 + jnp.exp(-h3))


def kernel(gene, disease, chromosome, variant, ref_base, mut_base, position,
           zygosity, allele_freq, aa_orig_props, aa_mut_props,
           gene_table, disease_table, chromosome_table, variant_table,
           base_table, fc1_w, fc1_b, fc2_w, fc2_b, fc3_w, fc3_b):
    f32 = jnp.float32
    mesh = plsc.VectorSubcoreMesh(core_axis_name="c", subcore_axis_name="s")

    gi = gene.reshape(NW, NCH, CHUNK)
    di = disease.reshape(NW, NCH, CHUNK)
    sc2 = pl.kernel(
        _sc_gather2_body,
        out_type=(jax.ShapeDtypeStruct((B, 128), f32),
                  jax.ShapeDtypeStruct((B, 128), f32)),
        mesh=mesh,
        scratch_types=(pltpu.VMEM((NCH, CHUNK), jnp.int32),
                       pltpu.VMEM((BPW, 128), f32),
                       pltpu.SemaphoreType.DMA),
    )
    gene_rows, dis_rows = sc2(gi, di, gene_table, disease_table)

    vi = variant.reshape(NW, BPW)
    var_t3 = variant_table.T.reshape(2, 8, 1000000)
    scv = pl.kernel(
        _sc_gatherv_body,
        out_type=jax.ShapeDtypeStruct((B * VDIM,), f32),
        mesh=mesh,
        scratch_types=(pltpu.VMEM((BPW,), jnp.int32),
                       pltpu.VMEM((2, 8, VB * 128), f32),
                       pltpu.VMEM((BPW * VDIM,), f32),
                       pltpu.SemaphoreType.DMA),
        compiler_params=pltpu.CompilerParams(needs_layout_passes=False),
    )
    var_rows = scv(vi, var_t3, gene_rows[:8]).reshape(B, VDIM)

    w1 = fc1_w.T
    w1g, w1d, w1c = w1[0:128], w1[128:256], w1[256:320]
    w1v, w1r, w1m, w1x = w1[320:336], w1[336:352], w1[352:368], w1[368:419]
    b1 = fc1_b.reshape(1, 128)
    w2 = fc2_w.T
    b2 = fc2_b.reshape(1, 64)
    w3 = fc3_w.T
    b3 = fc3_b.reshape(1, 1)
    afT = allele_freq.T
    aaoT = aa_orig_props.T
    aamT = aa_mut_props.T

    grid = (B // BLK,)

    def row(d):
        return pl.BlockSpec((BLK, d), lambda i: (i, 0))

    def col(d):
        return pl.BlockSpec((d, BLK), lambda i: (0, i))

    def vec():
        return pl.BlockSpec((BLK,), lambda i: (i,))

    def full(shape):
        return pl.BlockSpec(shape, lambda i: (0,) * len(shape))

    h1p = pl.pallas_call(
        _mlp1_body,
        grid=grid,
        in_specs=[row(128), row(128),
                  vec(), vec(), vec(), vec(), vec(),
                  col(9), col(20), col(20),
                  full((25, 64)), full((5, 16)),
                  full((128, 128)), full((128, 128)), full((64, 128)),
                  full((16, 128)), full((16, 128)), full((51, 128)),
                  full((1, 128))],
        out_specs=pl.BlockSpec((BLK, 128), lambda i: (i, 0)),
        out_shape=jax.ShapeDtypeStruct((B, 128), f32),
    )(gene_rows, dis_rows, chromosome, ref_base, mut_base,
      position, zygosity, afT, aaoT, aamT,
      chromosome_table, base_table,
      w1g, w1d, w1c, w1r, w1m, w1x, b1)

    outT = pl.pallas_call(
        _mlp2_body,
        grid=(B // BLK2,),
        in_specs=[pl.BlockSpec((BLK2, 128), lambda i: (i, 0)),
                  pl.BlockSpec((BLK2, VDIM), lambda i: (i, 0)),
                  full((16, 128)), full((128, 64)), full((1, 64)),
                  full((64, 1)), full((1, 1))],
        out_specs=pl.BlockSpec((1, BLK2), lambda i: (0, i)),
        out_shape=jax.ShapeDtypeStruct((1, B), f32),
    )(h1p, var_rows, w1v, w2, b2, w3, b3)
    return outT.reshape(B, 1)


# speedup vs baseline: 4.4583x; 1.1000x over previous
import jax
import jax.numpy as jnp
from jax import lax
from jax.experimental import pallas as pl
from jax.experimental.pallas import tpu as pltpu
from jax.experimental.pallas import tpu_sc as plsc

B = 16384
NC = 2
NS = 16
NW = NC * NS
BPW = B // NW
CHUNK = 128
NCH = BPW // CHUNK
VDIM = 16
BLK = 512


def _sc_gather2_body(gene_i, dis_i, gene_t, dis_t,
                     out_g, out_d, idxv, rows, sem):
    wid = lax.axis_index("s") * NC + lax.axis_index("c")
    base = wid * BPW
    for idx_hbm, tbl, out in ((gene_i, gene_t, out_g),
                              (dis_i, dis_t, out_d)):
        pltpu.sync_copy(idx_hbm.at[wid], idxv)
        cps = [pltpu.async_copy(tbl.at[idxv.at[j]],
                                rows.at[pl.ds(j * CHUNK, CHUNK)], sem)
               for j in range(NCH)]
        for cp in cps:
            cp.wait()
        pltpu.sync_copy(rows, out.at[pl.ds(base, BPW)])


VG = 16
VB = 32


def _sc_gatherv_body(var_i, var_t3, gdep, out_v, vidx, vbuf, rows16, sem):
    wid = lax.axis_index("s") * NC + lax.axis_index("c")
    base = wid * BPW
    pltpu.sync_copy(var_i.at[wid], vidx)
    iota16 = lax.iota(jnp.int32, 16)
    r_vec = iota16 >> 3
    s_vec = iota16 & 7

    for b in range(BPW // VB):
        b0 = b * VB

        def fetch(g, c):
            vvec = vidx[pl.ds(b0 + g * VG, VG)]
            for u in range(VG):
                off = pl.multiple_of((vvec[u] >> 7) * 128, 128)
                pltpu.async_copy(
                    var_t3.at[:, :, pl.ds(off, 128)],
                    vbuf.at[:, :, pl.ds((g * VG + u) * 128, 128)], sem)
            return c

        lax.fori_loop(0, VB // VG, fetch, 0)
        pltpu.make_async_copy(var_t3.at[:, :, pl.ds(0, VB * 128)],
                              vbuf, sem).wait()

        def extract(g, c):
            vvec = vidx[pl.ds(b0 + g * VG, VG)]
            for u in range(VG):
                i = b0 + g * VG + u
                lane = (g * VG + u) * 128 + (vvec[u] & 127)
                lane_vec = jnp.broadcast_to(lane, (16,))
                vals = plsc.load_gather(vbuf, [r_vec, s_vec, lane_vec])
                plsc.store_scatter(rows16, [i * VDIM + iota16], vals)
            return c

        lax.fori_loop(0, VB // VG, extract, 0)

    pltpu.sync_copy(rows16, out_v.at[pl.ds(base * VDIM, BPW * VDIM)])


def _leaky(x):
    return jnp.where(x >= 0, x, 0.01 * x)


def _mlp1_body(gene_r, dis_r, chrom_r, refb_r, mutb_r,
               pos_r, zyg_r, afT_r, aaoT_r, aamT_r,
               ct_r, bt_r,
               w1g_r, w1d_r, w1c_r, w1r_r, w1m_r, w1x_r,
               b1_r, h1p_r):
    f32 = jnp.float32

    def tdot(a, b):
        return lax.dot_general(a, b, (((0,), (0,)), ((), ())),
                               preferred_element_type=f32)

    ct_proj = jnp.dot(ct_r[...], w1c_r[...], preferred_element_type=f32)
    btp_r = jnp.dot(bt_r[...], w1r_r[...], preferred_element_type=f32)
    btp_m = jnp.dot(bt_r[...], w1m_r[...], preferred_element_type=f32)

    i25 = lax.broadcasted_iota(jnp.int32, (25, BLK), 0)
    i5 = lax.broadcasted_iota(jnp.int32, (5, BLK), 0)
    oh_c = (jnp.broadcast_to(chrom_r[...], (25, BLK)) == i25).astype(f32)
    oh_r = (jnp.broadcast_to(refb_r[...], (5, BLK)) == i5).astype(f32)
    oh_m = (jnp.broadcast_to(mutb_r[...], (5, BLK)) == i5).astype(f32)

    xdT = jnp.concatenate([pos_r[...].reshape(1, BLK),
                           zyg_r[...].reshape(1, BLK),
                           afT_r[...], aaoT_r[...], aamT_r[...]], axis=0)

    h1 = jnp.dot(gene_r[...], w1g_r[...], preferred_element_type=f32)
    h1 += jnp.dot(dis_r[...], w1d_r[...], preferred_element_type=f32)
    h1 += tdot(oh_c, ct_proj)
    h1 += tdot(oh_r, btp_r)
    h1 += tdot(oh_m, btp_m)
    h1 += tdot(xdT, w1x_r[...])
    h1p_r[...] = h1 + b1_r[...]


BLK2 = 2048


def _mlp2_body(h1p_r, var_r, w1v_r, w2_r, b2_r, w3_r, b3_r, out_r):
    f32 = jnp.float32
    h1 = _leaky(h1p_r[...] +
                jnp.dot(var_r[...], w1v_r[...], preferred_element_type=f32))
    h2 = _leaky(jnp.dot(h1, w2_r[...], preferred_element_type=f32) + b2_r[...])
    h3 = lax.dot_general(w3_r[...], h2, (((0,), (1,)), ((), ())),
                         preferred_element_type=f32) + b3_r[...]
    out_r[...] = 1.0 / (1.0 + jnp.exp(-h3))


def kernel(gene, disease, chromosome, variant, ref_base, mut_base, position,
           zygosity, allele_freq, aa_orig_props, aa_mut_props,
           gene_table, disease_table, chromosome_table, variant_table,
           base_table, fc1_w, fc1_b, fc2_w, fc2_b, fc3_w, fc3_b):
    f32 = jnp.float32
    mesh = plsc.VectorSubcoreMesh(core_axis_name="c", subcore_axis_name="s")

    gi = gene.reshape(NW, NCH, CHUNK)
    di = disease.reshape(NW, NCH, CHUNK)
    sc2 = pl.kernel(
        _sc_gather2_body,
        out_type=(jax.ShapeDtypeStruct((B, 128), f32),
                  jax.ShapeDtypeStruct((B, 128), f32)),
        mesh=mesh,
        scratch_types=(pltpu.VMEM((NCH, CHUNK), jnp.int32),
                       pltpu.VMEM((BPW, 128), f32),
                       pltpu.SemaphoreType.DMA),
    )
    gene_rows, dis_rows = sc2(gi, di, gene_table, disease_table)

    vi = variant.reshape(NW, BPW)
    var_t3 = variant_table.T.reshape(2, 8, 1000000)
    scv = pl.kernel(
        _sc_gatherv_body,
        out_type=jax.ShapeDtypeStruct((B * VDIM,), f32),
        mesh=mesh,
        scratch_types=(pltpu.VMEM((BPW,), jnp.int32),
                       pltpu.VMEM((2, 8, VB * 128), f32),
                       pltpu.VMEM((BPW * VDIM,), f32),
                       pltpu.SemaphoreType.DMA),
        compiler_params=pltpu.CompilerParams(needs_layout_passes=False),
    )
    var_rows = scv(vi, var_t3, gene_rows[:8]).reshape(B, VDIM)

    w1 = fc1_w.T
    w1g, w1d, w1c = w1[0:128], w1[128:256], w1[256:320]
    w1v, w1r, w1m, w1x = w1[320:336], w1[336:352], w1[352:368], w1[368:419]
    b1 = fc1_b.reshape(1, 128)
    w2 = fc2_w.T
    b2 = fc2_b.reshape(1, 64)
    w3 = fc3_w.T
    b3 = fc3_b.reshape(1, 1)
    afT = allele_freq.T
    aaoT = aa_orig_props.T
    aamT = aa_mut_props.T

    grid = (B // BLK,)

    def row(d):
        return pl.BlockSpec((BLK, d), lambda i: (i, 0))

    def col(d):
        return pl.BlockSpec((d, BLK), lambda i: (0, i))

    def vec():
        return pl.BlockSpec((BLK,), lambda i: (i,))

    def full(shape):
        return pl.BlockSpec(shape, lambda i: (0,) * len(shape))

    h1p = pl.pallas_call(
        _mlp1_body,
        grid=grid,
        in_specs=[row(128), row(128),
                  vec(), vec(), vec(), vec(), vec(),
                  col(9), col(20), col(20),
                  full((25, 64)), full((5, 16)),
                  full((128, 128)), full((128, 128)), full((64, 128)),
                  full((16, 128)), full((16, 128)), full((51, 128)),
                  full((1, 128))],
        out_specs=pl.BlockSpec((BLK, 128), lambda i: (i, 0)),
        out_shape=jax.ShapeDtypeStruct((B, 128), f32),
    )(gene_rows, dis_rows, chromosome, ref_base, mut_base,
      position, zygosity, afT, aaoT, aamT,
      chromosome_table, base_table,
      w1g, w1d, w1c, w1r, w1m, w1x, b1)

    outT = pl.pallas_call(
        _mlp2_body,
        grid=(B // BLK2,),
        in_specs=[pl.BlockSpec((BLK2, 128), lambda i: (i, 0)),
                  pl.BlockSpec((BLK2, VDIM), lambda i: (i, 0)),
                  full((16, 128)), full((128, 64)), full((1, 64)),
                  full((64, 1)), full((1, 1))],
        out_specs=pl.BlockSpec((1, BLK2), lambda i: (0, i)),
        out_shape=jax.ShapeDtypeStruct((1, B), f32),
    )(h1p, var_rows, w1v, w2, b2, w3, b3)
    return outT.reshape(B, 1)


# speedup vs baseline: 4.4765x; 1.0041x over previous
import jax
import jax.numpy as jnp
from jax import lax
from jax.experimental import pallas as pl
from jax.experimental.pallas import tpu as pltpu
from jax.experimental.pallas import tpu_sc as plsc

B = 16384
NC = 2
NS = 16
NW = NC * NS
BPW = B // NW
CHUNK = 128
NCH = BPW // CHUNK
VDIM = 16
BLK = 512


def _sc_gather2_body(gene_i, dis_i, gene_t, dis_t,
                     out_g, out_d, idxv, rows, sem):
    wid = lax.axis_index("s") * NC + lax.axis_index("c")
    base = wid * BPW
    for idx_hbm, tbl, out in ((gene_i, gene_t, out_g),
                              (dis_i, dis_t, out_d)):
        pltpu.sync_copy(idx_hbm.at[wid], idxv)
        cps = [pltpu.async_copy(tbl.at[idxv.at[j]],
                                rows.at[pl.ds(j * CHUNK, CHUNK)], sem)
               for j in range(NCH)]
        for cp in cps:
            cp.wait()
        pltpu.sync_copy(rows, out.at[pl.ds(base, BPW)])


VG = 16
VB = 32


def _sc_gatherv_body(var_i, var_t3, gdep, out_v, vidx, vbuf, rows16, sem):
    wid = lax.axis_index("s") * NC + lax.axis_index("c")
    base = wid * BPW
    pltpu.sync_copy(var_i.at[wid], vidx)
    iota16 = lax.iota(jnp.int32, 16)
    r_vec = iota16 >> 3
    s_vec = iota16 & 7

    for b in range(BPW // VB):
        b0 = b * VB

        def fetch(g, c):
            vvec = vidx[pl.ds(b0 + g * VG, VG)]
            for u in range(VG):
                off = pl.multiple_of((vvec[u] >> 7) * 128, 128)
                pltpu.async_copy(
                    var_t3.at[:, :, pl.ds(off, 128)],
                    vbuf.at[:, :, pl.ds((g * VG + u) * 128, 128)], sem)
            return c

        lax.fori_loop(0, VB // VG, fetch, 0)
        pltpu.make_async_copy(var_t3.at[:, :, pl.ds(0, VB * 128)],
                              vbuf, sem).wait()

        def extract(g, c):
            vvec = vidx[pl.ds(b0 + g * VG, VG)]
            for u in range(VG):
                i = b0 + g * VG + u
                lane = (g * VG + u) * 128 + (vvec[u] & 127)
                lane_vec = jnp.broadcast_to(lane, (16,))
                vals = plsc.load_gather(vbuf, [r_vec, s_vec, lane_vec])
                plsc.store_scatter(rows16, [i * VDIM + iota16], vals)
            return c

        lax.fori_loop(0, VB // VG, extract, 0)

    pltpu.sync_copy(rows16, out_v.at[pl.ds(base * VDIM, BPW * VDIM)])


def _leaky(x):
    return jnp.where(x >= 0, x, 0.01 * x)


def _mlp1_body(gene_r, dis_r, chrom_r, refb_r, mutb_r,
               pos_r, zyg_r, afT_r, aaoT_r, aamT_r,
               ct_r, bt_r,
               w1g_r, w1d_r, w1c_r, w1r_r, w1m_r, w1x_r,
               b1_r, h1p_r):
    f32 = jnp.float32

    def tdot(a, b):
        return lax.dot_general(a, b, (((0,), (0,)), ((), ())),
                               preferred_element_type=f32)

    ct_proj = jnp.dot(ct_r[...], w1c_r[...], preferred_element_type=f32)
    btp_r = jnp.dot(bt_r[...], w1r_r[...], preferred_element_type=f32)
    btp_m = jnp.dot(bt_r[...], w1m_r[...], preferred_element_type=f32)

    i25 = lax.broadcasted_iota(jnp.int32, (25, BLK), 0)
    i5 = lax.broadcasted_iota(jnp.int32, (5, BLK), 0)
    oh_c = (jnp.broadcast_to(chrom_r[...], (25, BLK)) == i25).astype(f32)
    oh_r = (jnp.broadcast_to(refb_r[...], (5, BLK)) == i5).astype(f32)
    oh_m = (jnp.broadcast_to(mutb_r[...], (5, BLK)) == i5).astype(f32)

    xdT = jnp.concatenate([pos_r[...].reshape(1, BLK),
                           zyg_r[...].reshape(1, BLK),
                           afT_r[...], aaoT_r[...], aamT_r[...]], axis=0)

    h1 = jnp.dot(gene_r[...], w1g_r[...], preferred_element_type=f32)
    h1 += jnp.dot(dis_r[...], w1d_r[...], preferred_element_type=f32)
    h1 += tdot(oh_c, ct_proj)
    h1 += tdot(oh_r, btp_r)
    h1 += tdot(oh_m, btp_m)
    h1 += tdot(xdT, w1x_r[...])
    h1p_r[...] = h1 + b1_r[...]


BLK2 = 2048


def _mlp2_body(h1p_r, var_r, w1v_r, w2_r, b2_r, w3_r, b3_r, out_r):
    f32 = jnp.float32
    h1 = _leaky(h1p_r[...] +
                jnp.dot(var_r[...], w1v_r[...], preferred_element_type=f32))
    h2 = _leaky(jnp.dot(h1, w2_r[...], preferred_element_type=f32) + b2_r[...])
    h3 = lax.dot_general(w3_r[...], h2, (((0,), (1,)), ((), ())),
                         preferred_element_type=f32) + b3_r[...]
    out_r[...] = (1.0 / (1.0 + jnp.exp(-h3)))[0]


def kernel(gene, disease, chromosome, variant, ref_base, mut_base, position,
           zygosity, allele_freq, aa_orig_props, aa_mut_props,
           gene_table, disease_table, chromosome_table, variant_table,
           base_table, fc1_w, fc1_b, fc2_w, fc2_b, fc3_w, fc3_b):
    f32 = jnp.float32
    mesh = plsc.VectorSubcoreMesh(core_axis_name="c", subcore_axis_name="s")

    gi = gene.reshape(NW, NCH, CHUNK)
    di = disease.reshape(NW, NCH, CHUNK)
    sc2 = pl.kernel(
        _sc_gather2_body,
        out_type=(jax.ShapeDtypeStruct((B, 128), f32),
                  jax.ShapeDtypeStruct((B, 128), f32)),
        mesh=mesh,
        scratch_types=(pltpu.VMEM((NCH, CHUNK), jnp.int32),
                       pltpu.VMEM((BPW, 128), f32),
                       pltpu.SemaphoreType.DMA),
    )
    gene_rows, dis_rows = sc2(gi, di, gene_table, disease_table)

    vi = variant.reshape(NW, BPW)
    var_t3 = variant_table.T.reshape(2, 8, 1000000)
    scv = pl.kernel(
        _sc_gatherv_body,
        out_type=jax.ShapeDtypeStruct((B * VDIM,), f32),
        mesh=mesh,
        scratch_types=(pltpu.VMEM((BPW,), jnp.int32),
                       pltpu.VMEM((2, 8, VB * 128), f32),
                       pltpu.VMEM((BPW * VDIM,), f32),
                       pltpu.SemaphoreType.DMA),
        compiler_params=pltpu.CompilerParams(needs_layout_passes=False),
    )
    var_rows = scv(vi, var_t3, gene_rows[:8]).reshape(B, VDIM)

    w1 = fc1_w.T
    w1g, w1d, w1c = w1[0:128], w1[128:256], w1[256:320]
    w1v, w1r, w1m, w1x = w1[320:336], w1[336:352], w1[352:368], w1[368:419]
    b1 = fc1_b.reshape(1, 128)
    w2 = fc2_w.T
    b2 = fc2_b.reshape(1, 64)
    w3 = fc3_w.T
    b3 = fc3_b.reshape(1, 1)
    afT = allele_freq.T
    aaoT = aa_orig_props.T
    aamT = aa_mut_props.T

    grid = (B // BLK,)

    def row(d):
        return pl.BlockSpec((BLK, d), lambda i: (i, 0))

    def col(d):
        return pl.BlockSpec((d, BLK), lambda i: (0, i))

    def vec():
        return pl.BlockSpec((BLK,), lambda i: (i,))

    def full(shape):
        return pl.BlockSpec(shape, lambda i: (0,) * len(shape))

    h1p = pl.pallas_call(
        _mlp1_body,
        grid=grid,
        in_specs=[row(128), row(128),
                  vec(), vec(), vec(), vec(), vec(),
                  col(9), col(20), col(20),
                  full((25, 64)), full((5, 16)),
                  full((128, 128)), full((128, 128)), full((64, 128)),
                  full((16, 128)), full((16, 128)), full((51, 128)),
                  full((1, 128))],
        out_specs=pl.BlockSpec((BLK, 128), lambda i: (i, 0)),
        out_shape=jax.ShapeDtypeStruct((B, 128), f32),
    )(gene_rows, dis_rows, chromosome, ref_base, mut_base,
      position, zygosity, afT, aaoT, aamT,
      chromosome_table, base_table,
      w1g, w1d, w1c, w1r, w1m, w1x, b1)

    outT = pl.pallas_call(
        _mlp2_body,
        grid=(B // BLK2,),
        in_specs=[pl.BlockSpec((BLK2, 128), lambda i: (i, 0)),
                  pl.BlockSpec((BLK2, VDIM), lambda i: (i, 0)),
                  full((16, 128)), full((128, 64)), full((1, 64)),
                  full((64, 1)), full((1, 1))],
        out_specs=pl.BlockSpec((BLK2,), lambda i: (i,)),
        out_shape=jax.ShapeDtypeStruct((B,), f32),
    )(h1p, var_rows, w1v, w2, b2, w3, b3)
    return outT.reshape(B, 1)
